# baseline (speedup 1.0000x reference)
import functools

import jax
import jax.numpy as jnp
from jax import lax
from jax.experimental import pallas as pl
from jax.experimental.pallas import tpu as pltpu
from jax.experimental.pallas import tpu_sc as plsc

_N = 10000
_E = 320000
_D = 128
_ALPHA = 0.1

_NC = 2
_NS = 16
_NW = _NC * _NS
_NP = 10240
_ROWS_T = _NP // _NS

_CH = 50
_CPT = _E // (_NW * _CH)
_NSEG = 5
_SEG = _CPT // _NSEG
_NB = 5

_CHD = 125
_CPTD = _E // (_NW * _CHD)

_RB = 1000

_mesh = plsc.VectorSubcoreMesh(core_axis_name="c", subcore_axis_name="s")


@functools.partial(
    pl.kernel,
    out_type=jax.ShapeDtypeStruct((_NC, _NP), jnp.float32),
    mesh=_mesh,
    scratch_types=[
        pltpu.VMEM((_CPTD, _CHD), jnp.int32),
        pltpu.VMEM((128,), jnp.float32),
        pltpu.VMEM((_ROWS_T,), jnp.float32),
        pltpu.VMEM_SHARED((_NP,), jnp.float32),
    ],
)
def _deg_kernel(dst_hbm, deg_hbm, idx_v, ones_v, z_v, deg_sp):
    c = lax.axis_index("c")
    s = lax.axis_index("s")

    def _zero(i, _):
        z_v[pl.ds(i * 16, 16)] = jnp.zeros((16,), jnp.float32)
        return 0

    lax.fori_loop(0, _ROWS_T // 16, _zero, 0)

    def _one(i, _):
        ones_v[pl.ds(i * 16, 16)] = jnp.ones((16,), jnp.float32)
        return 0

    lax.fori_loop(0, 8, _one, 0)

    pltpu.sync_copy(z_v, deg_sp.at[pl.ds(s * _ROWS_T, _ROWS_T)])
    base = (c * _NS + s) * _CPTD
    pltpu.sync_copy(dst_hbm.at[pl.ds(base, _CPTD)], idx_v)
    plsc.subcore_barrier()

    def _scat(j, _):
        pltpu.sync_copy(ones_v.at[pl.ds(0, _CHD)],
                        deg_sp.at[idx_v.at[j]], add=True)
        return 0

    lax.fori_loop(0, _CPTD, _scat, 0)
    plsc.subcore_barrier()
    pltpu.sync_copy(deg_sp.at[pl.ds(s * _ROWS_T, _ROWS_T)],
                    deg_hbm.at[c, pl.ds(s * _ROWS_T, _ROWS_T)])


@functools.partial(
    pl.kernel,
    out_type=jax.ShapeDtypeStruct((_NC, _NP, _D), jnp.float32),
    mesh=_mesh,
    scratch_types=[
        pltpu.VMEM((_SEG, _CH), jnp.int32),
        pltpu.VMEM((_SEG, _CH), jnp.int32),
        pltpu.VMEM((_CH, _D), jnp.float32),
        pltpu.VMEM((_CH, _D), jnp.float32),
        pltpu.VMEM((_CH, _D), jnp.float32),
        pltpu.VMEM((_CH, _D), jnp.float32),
        pltpu.VMEM((_CH, _D), jnp.float32),
        pltpu.VMEM_SHARED((_NP, _D), jnp.float32),
        pltpu.SemaphoreType.DMA,
        pltpu.SemaphoreType.DMA,
        pltpu.SemaphoreType.DMA,
        pltpu.SemaphoreType.DMA,
        pltpu.SemaphoreType.DMA,
        pltpu.SemaphoreType.DMA,
        pltpu.SemaphoreType.DMA,
        pltpu.SemaphoreType.DMA,
        pltpu.SemaphoreType.DMA,
        pltpu.SemaphoreType.DMA,
    ],
)
def _prop_kernel(hs_hbm, src_hbm, dst_hbm, zero_hbm, out_hbm,
                 src_v, dst_v, r0, r1, r2, r3, r4, agg_sp,
                 g0, g1, g2, g3, g4, s0, s1, s2, s3, s4):
    c = lax.axis_index("c")
    s = lax.axis_index("s")
    w = c * _NS + s
    rows = (r0, r1, r2, r3, r4)
    gsem = (g0, g1, g2, g3, g4)
    ssem = (s0, s1, s2, s3, s4)

    pltpu.sync_copy(zero_hbm.at[pl.ds(s * _ROWS_T, _ROWS_T)],
                    agg_sp.at[pl.ds(s * _ROWS_T, _ROWS_T)])
    plsc.subcore_barrier()

    def _gather(j, b):
        pltpu.async_copy(hs_hbm.at[src_v.at[j]], rows[b], gsem[b])

    def _scatter(j, b):
        pltpu.async_copy(rows[b], agg_sp.at[dst_v.at[j]], ssem[b], add=True)

    def _wait_g(b):
        pltpu.make_async_copy(hs_hbm.at[src_v.at[0]], rows[b], gsem[b]).wait()

    def _wait_s(b):
        pltpu.make_async_copy(rows[b], agg_sp.at[dst_v.at[0]], ssem[b]).wait()

    for seg in range(_NSEG):
        pltpu.sync_copy(src_hbm.at[w, pl.ds(seg * _SEG, _SEG)], src_v)
        pltpu.sync_copy(dst_hbm.at[w, pl.ds(seg * _SEG, _SEG)], dst_v)

        _gather(0, 0)
        _gather(1, 1)
        _wait_g(0); _scatter(0, 0); _gather(2, 2)
        _wait_g(1); _scatter(1, 1); _gather(3, 3)
        _wait_g(2); _scatter(2, 2); _gather(4, 4)
        _wait_g(3); _scatter(3, 3); _wait_s(0); _gather(5, 0)
        _wait_g(4); _scatter(4, 4); _wait_s(1); _gather(6, 1)

        def _body(it, _):
            j0 = 5 * it
            for k in range(5):
                j = j0 + k
                b = k
                nb = (k + 2) % 5
                _wait_g(b)
                _scatter(j, b)
                _wait_s(nb)
                jn = jnp.minimum(j + 2, _SEG - 1)
                _gather(jn, nb)
            return 0

        lax.fori_loop(1, _SEG // 5, _body, 0)
        _wait_g(0)
        _wait_g(1)
        _wait_s(2)
        _wait_s(3)
        _wait_s(4)

    plsc.subcore_barrier()
    pltpu.sync_copy(agg_sp.at[pl.ds(s * _ROWS_T, _ROWS_T)],
                    out_hbm.at[c, pl.ds(s * _ROWS_T, _ROWS_T)])


def _prep_body(d0_ref, d1_ref, x0_ref, hs_ref):
    rinv = lax.rsqrt(jnp.maximum(d0_ref[0] + d1_ref[0], 1.0))
    hs_ref[...] = x0_ref[...] * rinv


_prep_call = pl.pallas_call(
    _prep_body,
    grid=(_N // _RB,),
    in_specs=[
        pl.BlockSpec((1, _RB, 1), lambda i: (0, i, 0)),
        pl.BlockSpec((1, _RB, 1), lambda i: (1, i, 0)),
        pl.BlockSpec((_RB, _D), lambda i: (i, 0)),
    ],
    out_specs=pl.BlockSpec((_RB, _D), lambda i: (i, 0)),
    out_shape=jax.ShapeDtypeStruct((_N, _D), jnp.float32),
)


def _dense_body(p0_ref, p1_ref, d0_ref, d1_ref, x0_ref, w_ref, b_ref,
                out_ref, hs_ref):
    rinv = lax.rsqrt(jnp.maximum(d0_ref[0] + d1_ref[0], 1.0))
    agg = (p0_ref[0] + p1_ref[0]) * rinv
    support = (1.0 - _ALPHA) * agg + _ALPHA * x0_ref[...]
    o = jnp.dot(support, w_ref[...], preferred_element_type=jnp.float32)
    o = jnp.maximum(o + b_ref[...], 0.0)
    out_ref[...] = o
    hs_ref[...] = o * rinv


_dense_call = pl.pallas_call(
    _dense_body,
    grid=(_N // _RB,),
    in_specs=[
        pl.BlockSpec((1, _RB, _D), lambda i: (0, i, 0)),
        pl.BlockSpec((1, _RB, _D), lambda i: (1, i, 0)),
        pl.BlockSpec((1, _RB, 1), lambda i: (0, i, 0)),
        pl.BlockSpec((1, _RB, 1), lambda i: (1, i, 0)),
        pl.BlockSpec((_RB, _D), lambda i: (i, 0)),
        pl.BlockSpec((_D, _D), lambda i: (0, 0)),
        pl.BlockSpec((1, _D), lambda i: (0, 0)),
    ],
    out_specs=[
        pl.BlockSpec((_RB, _D), lambda i: (i, 0)),
        pl.BlockSpec((_RB, _D), lambda i: (i, 0)),
    ],
    out_shape=[
        jax.ShapeDtypeStruct((_N, _D), jnp.float32),
        jax.ShapeDtypeStruct((_N, _D), jnp.float32),
    ],
)


def kernel(drug_cell_pair_feature, edge_idx, W, b):
    x0 = drug_cell_pair_feature
    src = edge_idx[0].astype(jnp.int32)
    dst = edge_idx[1].astype(jnp.int32)
    src3 = src.reshape(_NW, _CPT, _CH)
    dst3 = dst.reshape(_NW, _CPT, _CH)
    dst2d = dst.reshape(_NW * _CPTD, _CHD)
    b2 = b.reshape(1, _D)
    zeros = jnp.zeros((_NP, _D), jnp.float32)

    deg = _deg_kernel(dst2d)
    deg3 = deg.reshape(_NC, _NP, 1)
    hs = _prep_call(deg3, deg3, x0)
    out = x0
    for _ in range(5):
        p = _prop_kernel(hs, src3, dst3, zeros)
        out, hs = _dense_call(p, p, deg3, deg3, x0, W, b2)
    return out


# speedup vs baseline: 1.2180x; 1.2180x over previous
import functools

import jax
import jax.numpy as jnp
from jax import lax
from jax.experimental import pallas as pl
from jax.experimental.pallas import tpu as pltpu
from jax.experimental.pallas import tpu_sc as plsc

_N = 10000
_E = 320000
_D = 128
_ALPHA = 0.1

_NC = 2
_NS = 16
_NP = 10240
_ROWS_T = _NP // _NS

_CH = 125
_CPT = _E // (_NC * _NS * _CH)

_RB = 1000

_mesh = plsc.VectorSubcoreMesh(core_axis_name="c", subcore_axis_name="s")


@functools.partial(
    pl.kernel,
    out_type=jax.ShapeDtypeStruct((_NC, _NP), jnp.float32),
    mesh=_mesh,
    scratch_types=[
        pltpu.VMEM((_CPT, _CH), jnp.int32),
        pltpu.VMEM((128,), jnp.float32),
        pltpu.VMEM((_ROWS_T,), jnp.float32),
        pltpu.VMEM_SHARED((_NP,), jnp.float32),
        pltpu.SemaphoreType.DMA,
        pltpu.SemaphoreType.DMA,
    ],
)
def _deg_kernel(dst_hbm, deg_hbm, idx_v, ones_v, z_v, deg_sp, dsem_a, dsem_b):
    c = lax.axis_index("c")
    s = lax.axis_index("s")

    def _zero(i, _):
        z_v[pl.ds(i * 16, 16)] = jnp.zeros((16,), jnp.float32)
        return 0

    lax.fori_loop(0, _ROWS_T // 16, _zero, 0)

    def _one(i, _):
        ones_v[pl.ds(i * 16, 16)] = jnp.ones((16,), jnp.float32)
        return 0

    lax.fori_loop(0, 8, _one, 0)

    pltpu.sync_copy(z_v, deg_sp.at[pl.ds(s * _ROWS_T, _ROWS_T)])
    base = (c * _NS + s) * _CPT
    pltpu.sync_copy(dst_hbm.at[pl.ds(base, _CPT)], idx_v)
    plsc.subcore_barrier()

    def _scat(i, _):
        j0 = 2 * i
        pltpu.async_copy(ones_v.at[pl.ds(0, _CH)],
                         deg_sp.at[idx_v.at[j0]], dsem_a, add=True)
        pltpu.async_copy(ones_v.at[pl.ds(0, _CH)],
                         deg_sp.at[idx_v.at[j0 + 1]], dsem_b, add=True)
        pltpu.make_async_copy(ones_v.at[pl.ds(0, _CH)],
                              deg_sp.at[idx_v.at[j0]], dsem_a).wait()
        pltpu.make_async_copy(ones_v.at[pl.ds(0, _CH)],
                              deg_sp.at[idx_v.at[j0]], dsem_b).wait()
        return 0

    lax.fori_loop(0, _CPT // 2, _scat, 0)
    plsc.subcore_barrier()
    pltpu.sync_copy(deg_sp.at[pl.ds(s * _ROWS_T, _ROWS_T)],
                    deg_hbm.at[c, pl.ds(s * _ROWS_T, _ROWS_T)])


@functools.partial(
    pl.kernel,
    out_type=jax.ShapeDtypeStruct((_NC, _NP, _D), jnp.float32),
    mesh=_mesh,
    scratch_types=[
        pltpu.VMEM((_CPT // 2, _CH), jnp.int32),
        pltpu.VMEM((_CPT // 2, _CH), jnp.int32),
        pltpu.VMEM((_CH, _D), jnp.float32),
        pltpu.VMEM((_CH, _D), jnp.float32),
        pltpu.VMEM_SHARED((_NP, _D), jnp.float32),
        pltpu.SemaphoreType.DMA,
        pltpu.SemaphoreType.DMA,
        pltpu.SemaphoreType.DMA,
        pltpu.SemaphoreType.DMA,
    ],
)
def _prop_kernel(hs_hbm, src_hbm, dst_hbm, zero_hbm, out_hbm,
                 src_v, dst_v, rows_a, rows_b, agg_sp, sem_a, sem_b,
                 sem_a2, sem_b2):
    c = lax.axis_index("c")
    s = lax.axis_index("s")
    half = _CPT // 2

    pltpu.sync_copy(zero_hbm.at[pl.ds(s * _ROWS_T, _ROWS_T)],
                    agg_sp.at[pl.ds(s * _ROWS_T, _ROWS_T)])
    base = (c * _NS + s) * _CPT
    pltpu.sync_copy(src_hbm.at[pl.ds(base, half)], src_v)
    pltpu.sync_copy(dst_hbm.at[pl.ds(base, half)], dst_v)
    plsc.subcore_barrier()

    def _gath(j, buf, sx, sy):
        pltpu.async_copy(hs_hbm.at[src_v.at[j, pl.ds(0, 64)]],
                         buf.at[pl.ds(0, 64)], sx)
        pltpu.async_copy(hs_hbm.at[src_v.at[j, pl.ds(64, 61)]],
                         buf.at[pl.ds(64, 61)], sy)

    def _wg(buf, sx, sy):
        pltpu.make_async_copy(hs_hbm.at[src_v.at[0, pl.ds(0, 64)]],
                              buf.at[pl.ds(0, 64)], sx).wait()
        pltpu.make_async_copy(hs_hbm.at[src_v.at[0, pl.ds(64, 61)]],
                              buf.at[pl.ds(64, 61)], sy).wait()

    def _run_half(_):
        _gath(0, rows_a, sem_a, sem_a2)

        def _body(i, _):
            j0 = 2 * i
            _gath(j0 + 1, rows_b, sem_b, sem_b2)
            _wg(rows_a, sem_a, sem_a2)
            pltpu.sync_copy(rows_a, agg_sp.at[dst_v.at[j0]], add=True)
            jn = jnp.minimum(j0 + 2, half - 1)
            _gath(jn, rows_a, sem_a, sem_a2)
            _wg(rows_b, sem_b, sem_b2)
            pltpu.sync_copy(rows_b, agg_sp.at[dst_v.at[j0 + 1]], add=True)
            return 0

        lax.fori_loop(0, half // 2, _body, 0)
        _wg(rows_a, sem_a, sem_a2)

    _run_half(None)
    pltpu.sync_copy(src_hbm.at[pl.ds(base + half, half)], src_v)
    pltpu.sync_copy(dst_hbm.at[pl.ds(base + half, half)], dst_v)
    _run_half(None)
    plsc.subcore_barrier()
    pltpu.sync_copy(agg_sp.at[pl.ds(s * _ROWS_T, _ROWS_T)],
                    out_hbm.at[c, pl.ds(s * _ROWS_T, _ROWS_T)])


def _prep_body(d0_ref, d1_ref, x0_ref, hs_ref):
    rinv = lax.rsqrt(jnp.maximum(d0_ref[0] + d1_ref[0], 1.0))
    hs_ref[...] = x0_ref[...] * rinv


_prep_call = pl.pallas_call(
    _prep_body,
    grid=(_N // _RB,),
    in_specs=[
        pl.BlockSpec((1, _RB, 1), lambda i: (0, i, 0)),
        pl.BlockSpec((1, _RB, 1), lambda i: (1, i, 0)),
        pl.BlockSpec((_RB, _D), lambda i: (i, 0)),
    ],
    out_specs=pl.BlockSpec((_RB, _D), lambda i: (i, 0)),
    out_shape=jax.ShapeDtypeStruct((_N, _D), jnp.float32),
)


def _dense_mid_body(p0_ref, p1_ref, d0_ref, d1_ref, x0_ref, w_ref, b_ref,
                    hs_ref):
    rinv = lax.rsqrt(jnp.maximum(d0_ref[0] + d1_ref[0], 1.0))
    agg = (p0_ref[0] + p1_ref[0]) * rinv
    support = (1.0 - _ALPHA) * agg + _ALPHA * x0_ref[...]
    o = jnp.dot(support, w_ref[...], preferred_element_type=jnp.float32)
    o = jnp.maximum(o + b_ref[...], 0.0)
    hs_ref[...] = o * rinv


def _dense_fin_body(p0_ref, p1_ref, d0_ref, d1_ref, x0_ref, w_ref, b_ref,
                    out_ref):
    rinv = lax.rsqrt(jnp.maximum(d0_ref[0] + d1_ref[0], 1.0))
    agg = (p0_ref[0] + p1_ref[0]) * rinv
    support = (1.0 - _ALPHA) * agg + _ALPHA * x0_ref[...]
    o = jnp.dot(support, w_ref[...], preferred_element_type=jnp.float32)
    out_ref[...] = jnp.maximum(o + b_ref[...], 0.0)


_dense_in_specs = [
    pl.BlockSpec((1, _RB, _D), lambda i: (0, i, 0)),
    pl.BlockSpec((1, _RB, _D), lambda i: (1, i, 0)),
    pl.BlockSpec((1, _RB, 1), lambda i: (0, i, 0)),
    pl.BlockSpec((1, _RB, 1), lambda i: (1, i, 0)),
    pl.BlockSpec((_RB, _D), lambda i: (i, 0)),
    pl.BlockSpec((_D, _D), lambda i: (0, 0)),
    pl.BlockSpec((1, _D), lambda i: (0, 0)),
]

_dense_mid_call = pl.pallas_call(
    _dense_mid_body,
    grid=(_N // _RB,),
    in_specs=_dense_in_specs,
    out_specs=pl.BlockSpec((_RB, _D), lambda i: (i, 0)),
    out_shape=jax.ShapeDtypeStruct((_N, _D), jnp.float32),
)

_dense_fin_call = pl.pallas_call(
    _dense_fin_body,
    grid=(_N // _RB,),
    in_specs=_dense_in_specs,
    out_specs=pl.BlockSpec((_RB, _D), lambda i: (i, 0)),
    out_shape=jax.ShapeDtypeStruct((_N, _D), jnp.float32),
)


def kernel(drug_cell_pair_feature, edge_idx, W, b):
    x0 = drug_cell_pair_feature
    src = edge_idx[0].astype(jnp.int32)
    dst = edge_idx[1].astype(jnp.int32)
    src2d = src.reshape(_NC * _NS * _CPT, _CH)
    dst2d = dst.reshape(_NC * _NS * _CPT, _CH)
    b2 = b.reshape(1, _D)
    zeros = jnp.zeros((_NP, _D), jnp.float32)

    deg = _deg_kernel(dst2d)
    deg3 = deg.reshape(_NC, _NP, 1)
    hs = _prep_call(deg3, deg3, x0)
    for _ in range(4):
        p = _prop_kernel(hs, src2d, dst2d, zeros)
        hs = _dense_mid_call(p, p, deg3, deg3, x0, W, b2)
    p = _prop_kernel(hs, src2d, dst2d, zeros)
    return _dense_fin_call(p, p, deg3, deg3, x0, W, b2)


# speedup vs baseline: 1.2396x; 1.0178x over previous
import functools

import jax
import jax.numpy as jnp
from jax import lax
from jax.experimental import pallas as pl
from jax.experimental.pallas import tpu as pltpu
from jax.experimental.pallas import tpu_sc as plsc

_N = 10000
_E = 320000
_D = 128
_ALPHA = 0.1

_NC = 2
_NS = 16
_NP = 10240
_ROWS_T = _NP // _NS

_CH = 125
_CPT = _E // (_NC * _NS * _CH)

_RB = 1000

_mesh = plsc.VectorSubcoreMesh(core_axis_name="c", subcore_axis_name="s")


@functools.partial(
    pl.kernel,
    out_type=jax.ShapeDtypeStruct((_NC, _NP), jnp.float32),
    mesh=_mesh,
    scratch_types=[
        pltpu.VMEM((_CPT, _CH), jnp.int32),
        pltpu.VMEM((128,), jnp.float32),
        pltpu.VMEM((_ROWS_T,), jnp.float32),
        pltpu.VMEM_SHARED((_NP,), jnp.float32),
        pltpu.SemaphoreType.DMA,
        pltpu.SemaphoreType.DMA,
    ],
)
def _deg_kernel(dst_hbm, deg_hbm, idx_v, ones_v, z_v, deg_sp, dsem_a, dsem_b):
    c = lax.axis_index("c")
    s = lax.axis_index("s")

    def _zero(i, _):
        z_v[pl.ds(i * 16, 16)] = jnp.zeros((16,), jnp.float32)
        return 0

    lax.fori_loop(0, _ROWS_T // 16, _zero, 0)

    def _one(i, _):
        ones_v[pl.ds(i * 16, 16)] = jnp.ones((16,), jnp.float32)
        return 0

    lax.fori_loop(0, 8, _one, 0)

    pltpu.sync_copy(z_v, deg_sp.at[pl.ds(s * _ROWS_T, _ROWS_T)])
    base = (c * _NS + s) * _CPT
    pltpu.sync_copy(dst_hbm.at[pl.ds(base, _CPT)], idx_v)
    plsc.subcore_barrier()

    def _scat(i, _):
        j0 = 2 * i
        pltpu.async_copy(ones_v.at[pl.ds(0, _CH)],
                         deg_sp.at[idx_v.at[j0]], dsem_a, add=True)
        pltpu.async_copy(ones_v.at[pl.ds(0, _CH)],
                         deg_sp.at[idx_v.at[j0 + 1]], dsem_b, add=True)
        pltpu.make_async_copy(ones_v.at[pl.ds(0, _CH)],
                              deg_sp.at[idx_v.at[j0]], dsem_a).wait()
        pltpu.make_async_copy(ones_v.at[pl.ds(0, _CH)],
                              deg_sp.at[idx_v.at[j0]], dsem_b).wait()
        return 0

    lax.fori_loop(0, _CPT // 2, _scat, 0)
    plsc.subcore_barrier()
    pltpu.sync_copy(deg_sp.at[pl.ds(s * _ROWS_T, _ROWS_T)],
                    deg_hbm.at[c, pl.ds(s * _ROWS_T, _ROWS_T)])


@functools.partial(
    pl.kernel,
    out_type=jax.ShapeDtypeStruct((_NC, _NP, _D), jnp.float32),
    mesh=_mesh,
    scratch_types=[
        pltpu.VMEM((_CPT // 2, _CH), jnp.int32),
        pltpu.VMEM((_CPT // 2, _CH), jnp.int32),
        pltpu.VMEM((_CH, _D), jnp.float32),
        pltpu.VMEM((_CH, _D), jnp.float32),
        pltpu.VMEM_SHARED((_NP, _D), jnp.float32),
        pltpu.SemaphoreType.DMA,
        pltpu.SemaphoreType.DMA,
    ],
)
def _prop_kernel(hs_hbm, src_hbm, dst_hbm, zero_hbm, out_hbm,
                 src_v, dst_v, rows_a, rows_b, agg_sp, sem_a, sem_b):
    c = lax.axis_index("c")
    s = lax.axis_index("s")
    half = _CPT // 2

    pltpu.sync_copy(zero_hbm.at[pl.ds(s * _ROWS_T, _ROWS_T)],
                    agg_sp.at[pl.ds(s * _ROWS_T, _ROWS_T)])
    base = (c * _NS + s) * _CPT
    pltpu.sync_copy(src_hbm.at[pl.ds(base, half)], src_v)
    pltpu.sync_copy(dst_hbm.at[pl.ds(base, half)], dst_v)
    plsc.subcore_barrier()

    def _run_half(_):
        pltpu.async_copy(hs_hbm.at[src_v.at[0]], rows_a, sem_a)

        def _body(i, _):
            j0 = 2 * i
            pltpu.async_copy(hs_hbm.at[src_v.at[j0 + 1]], rows_b, sem_b)
            pltpu.make_async_copy(hs_hbm.at[src_v.at[j0]], rows_a, sem_a).wait()
            pltpu.sync_copy(rows_a, agg_sp.at[dst_v.at[j0]], add=True)
            jn = jnp.minimum(j0 + 2, half - 1)
            pltpu.async_copy(hs_hbm.at[src_v.at[jn]], rows_a, sem_a)
            pltpu.make_async_copy(hs_hbm.at[src_v.at[j0 + 1]], rows_b, sem_b).wait()
            pltpu.sync_copy(rows_b, agg_sp.at[dst_v.at[j0 + 1]], add=True)
            return 0

        lax.fori_loop(0, half // 2, _body, 0)
        pltpu.make_async_copy(hs_hbm.at[src_v.at[half - 1]], rows_a, sem_a).wait()

    _run_half(None)
    pltpu.sync_copy(src_hbm.at[pl.ds(base + half, half)], src_v)
    pltpu.sync_copy(dst_hbm.at[pl.ds(base + half, half)], dst_v)
    _run_half(None)
    plsc.subcore_barrier()
    pltpu.sync_copy(agg_sp.at[pl.ds(s * _ROWS_T, _ROWS_T)],
                    out_hbm.at[c, pl.ds(s * _ROWS_T, _ROWS_T)])


def _prep_body(d0_ref, d1_ref, x0_ref, hs_ref):
    rinv = lax.rsqrt(jnp.maximum(d0_ref[0] + d1_ref[0], 1.0))
    hs_ref[...] = x0_ref[...] * rinv


_prep_call = pl.pallas_call(
    _prep_body,
    grid=(_N // _RB,),
    in_specs=[
        pl.BlockSpec((1, _RB, 1), lambda i: (0, i, 0)),
        pl.BlockSpec((1, _RB, 1), lambda i: (1, i, 0)),
        pl.BlockSpec((_RB, _D), lambda i: (i, 0)),
    ],
    out_specs=pl.BlockSpec((_RB, _D), lambda i: (i, 0)),
    out_shape=jax.ShapeDtypeStruct((_N, _D), jnp.float32),
)


def _dense_mid_body(p0_ref, p1_ref, d0_ref, d1_ref, x0_ref, w_ref, b_ref,
                    hs_ref):
    rinv = lax.rsqrt(jnp.maximum(d0_ref[0] + d1_ref[0], 1.0))
    agg = (p0_ref[0] + p1_ref[0]) * rinv
    support = (1.0 - _ALPHA) * agg + _ALPHA * x0_ref[...]
    o = jnp.dot(support, w_ref[...], preferred_element_type=jnp.float32)
    o = jnp.maximum(o + b_ref[...], 0.0)
    hs_ref[...] = o * rinv


def _dense_fin_body(p0_ref, p1_ref, d0_ref, d1_ref, x0_ref, w_ref, b_ref,
                    out_ref):
    rinv = lax.rsqrt(jnp.maximum(d0_ref[0] + d1_ref[0], 1.0))
    agg = (p0_ref[0] + p1_ref[0]) * rinv
    support = (1.0 - _ALPHA) * agg + _ALPHA * x0_ref[...]
    o = jnp.dot(support, w_ref[...], preferred_element_type=jnp.float32)
    out_ref[...] = jnp.maximum(o + b_ref[...], 0.0)


_dense_in_specs = [
    pl.BlockSpec((1, _RB, _D), lambda i: (0, i, 0)),
    pl.BlockSpec((1, _RB, _D), lambda i: (1, i, 0)),
    pl.BlockSpec((1, _RB, 1), lambda i: (0, i, 0)),
    pl.BlockSpec((1, _RB, 1), lambda i: (1, i, 0)),
    pl.BlockSpec((_RB, _D), lambda i: (i, 0)),
    pl.BlockSpec((_D, _D), lambda i: (0, 0)),
    pl.BlockSpec((1, _D), lambda i: (0, 0)),
]

_dense_mid_call = pl.pallas_call(
    _dense_mid_body,
    grid=(_N // _RB,),
    in_specs=_dense_in_specs,
    out_specs=pl.BlockSpec((_RB, _D), lambda i: (i, 0)),
    out_shape=jax.ShapeDtypeStruct((_N, _D), jnp.float32),
)

_dense_fin_call = pl.pallas_call(
    _dense_fin_body,
    grid=(_N // _RB,),
    in_specs=_dense_in_specs,
    out_specs=pl.BlockSpec((_RB, _D), lambda i: (i, 0)),
    out_shape=jax.ShapeDtypeStruct((_N, _D), jnp.float32),
)


def kernel(drug_cell_pair_feature, edge_idx, W, b):
    x0 = drug_cell_pair_feature
    src = edge_idx[0].astype(jnp.int32)
    dst = edge_idx[1].astype(jnp.int32)
    src2d = src.reshape(_NC * _NS * _CPT, _CH)
    dst2d = dst.reshape(_NC * _NS * _CPT, _CH)
    b2 = b.reshape(1, _D)
    zeros = jnp.zeros((_NP, _D), jnp.float32)

    deg = _deg_kernel(dst2d)
    deg3 = deg.reshape(_NC, _NP, 1)
    hs = _prep_call(deg3, deg3, x0)
    for _ in range(4):
        p = _prop_kernel(hs, src2d, dst2d, zeros)
        hs = _dense_mid_call(p, p, deg3, deg3, x0, W, b2)
    p = _prop_kernel(hs, src2d, dst2d, zeros)
    return _dense_fin_call(p, p, deg3, deg3, x0, W, b2)


# speedup vs baseline: 1.2608x; 1.0171x over previous
import functools

import jax
import jax.numpy as jnp
from jax import lax
from jax.experimental import pallas as pl
from jax.experimental.pallas import tpu as pltpu
from jax.experimental.pallas import tpu_sc as plsc

_N = 10000
_E = 320000
_D = 128
_ALPHA = 0.1

_NC = 2
_NS = 16
_NP = 10240
_ROWS_T = _NP // _NS

_CH = 125
_CPT = _E // (_NC * _NS * _CH)

_RB = 2000

_mesh = plsc.VectorSubcoreMesh(core_axis_name="c", subcore_axis_name="s")


@functools.partial(
    pl.kernel,
    out_type=jax.ShapeDtypeStruct((_NC, _NP), jnp.float32),
    mesh=_mesh,
    scratch_types=[
        pltpu.VMEM((_CPT, _CH), jnp.int32),
        pltpu.VMEM((128,), jnp.float32),
        pltpu.VMEM((_ROWS_T,), jnp.float32),
        pltpu.VMEM_SHARED((_NP,), jnp.float32),
        pltpu.SemaphoreType.DMA,
        pltpu.SemaphoreType.DMA,
    ],
)
def _deg_kernel(dst_hbm, deg_hbm, idx_v, ones_v, z_v, deg_sp, dsem_a, dsem_b):
    c = lax.axis_index("c")
    s = lax.axis_index("s")

    def _zero(i, _):
        z_v[pl.ds(i * 16, 16)] = jnp.zeros((16,), jnp.float32)
        return 0

    lax.fori_loop(0, _ROWS_T // 16, _zero, 0)

    def _one(i, _):
        ones_v[pl.ds(i * 16, 16)] = jnp.ones((16,), jnp.float32)
        return 0

    lax.fori_loop(0, 8, _one, 0)

    pltpu.sync_copy(z_v, deg_sp.at[pl.ds(s * _ROWS_T, _ROWS_T)])
    base = (c * _NS + s) * _CPT
    pltpu.sync_copy(dst_hbm.at[pl.ds(base, _CPT)], idx_v)
    plsc.subcore_barrier()

    def _scat(i, _):
        j0 = 2 * i
        pltpu.async_copy(ones_v.at[pl.ds(0, _CH)],
                         deg_sp.at[idx_v.at[j0]], dsem_a, add=True)
        pltpu.async_copy(ones_v.at[pl.ds(0, _CH)],
                         deg_sp.at[idx_v.at[j0 + 1]], dsem_b, add=True)
        pltpu.make_async_copy(ones_v.at[pl.ds(0, _CH)],
                              deg_sp.at[idx_v.at[j0]], dsem_a).wait()
        pltpu.make_async_copy(ones_v.at[pl.ds(0, _CH)],
                              deg_sp.at[idx_v.at[j0]], dsem_b).wait()
        return 0

    lax.fori_loop(0, _CPT // 2, _scat, 0)
    plsc.subcore_barrier()
    pltpu.sync_copy(deg_sp.at[pl.ds(s * _ROWS_T, _ROWS_T)],
                    deg_hbm.at[c, pl.ds(s * _ROWS_T, _ROWS_T)])


@functools.partial(
    pl.kernel,
    out_type=jax.ShapeDtypeStruct((_NC, _NP, _D), jnp.float32),
    mesh=_mesh,
    scratch_types=[
        pltpu.VMEM((_CPT // 2, _CH), jnp.int32),
        pltpu.VMEM((_CPT // 2, _CH), jnp.int32),
        pltpu.VMEM((_CH, _D), jnp.float32),
        pltpu.VMEM((_CH, _D), jnp.float32),
        pltpu.VMEM_SHARED((_NP, _D), jnp.float32),
        pltpu.SemaphoreType.DMA,
        pltpu.SemaphoreType.DMA,
    ],
)
def _prop_kernel(hs_hbm, src_hbm, dst_hbm, zero_hbm, out_hbm,
                 src_v, dst_v, rows_a, rows_b, agg_sp, sem_a, sem_b):
    c = lax.axis_index("c")
    s = lax.axis_index("s")
    half = _CPT // 2

    pltpu.sync_copy(zero_hbm.at[pl.ds(s * _ROWS_T, _ROWS_T)],
                    agg_sp.at[pl.ds(s * _ROWS_T, _ROWS_T)])
    base = (c * _NS + s) * _CPT
    pltpu.sync_copy(src_hbm.at[pl.ds(base, half)], src_v)
    pltpu.sync_copy(dst_hbm.at[pl.ds(base, half)], dst_v)
    plsc.subcore_barrier()

    def _run_half(_):
        pltpu.async_copy(hs_hbm.at[src_v.at[0]], rows_a, sem_a)

        def _body(i, _):
            j0 = 2 * i
            pltpu.async_copy(hs_hbm.at[src_v.at[j0 + 1]], rows_b, sem_b)
            pltpu.make_async_copy(hs_hbm.at[src_v.at[j0]], rows_a, sem_a).wait()
            pltpu.sync_copy(rows_a, agg_sp.at[dst_v.at[j0]], add=True)
            jn = jnp.minimum(j0 + 2, half - 1)
            pltpu.async_copy(hs_hbm.at[src_v.at[jn]], rows_a, sem_a)
            pltpu.make_async_copy(hs_hbm.at[src_v.at[j0 + 1]], rows_b, sem_b).wait()
            pltpu.sync_copy(rows_b, agg_sp.at[dst_v.at[j0 + 1]], add=True)
            return 0

        lax.fori_loop(0, half // 2, _body, 0)
        pltpu.make_async_copy(hs_hbm.at[src_v.at[half - 1]], rows_a, sem_a).wait()

    _run_half(None)
    pltpu.sync_copy(src_hbm.at[pl.ds(base + half, half)], src_v)
    pltpu.sync_copy(dst_hbm.at[pl.ds(base + half, half)], dst_v)
    _run_half(None)
    plsc.subcore_barrier()
    pltpu.sync_copy(agg_sp.at[pl.ds(s * _ROWS_T, _ROWS_T)],
                    out_hbm.at[c, pl.ds(s * _ROWS_T, _ROWS_T)])


def _prep_body(d0_ref, d1_ref, x0_ref, hs_ref):
    rinv = lax.rsqrt(jnp.maximum(d0_ref[0] + d1_ref[0], 1.0))
    hs_ref[...] = x0_ref[...] * rinv


_prep_call = pl.pallas_call(
    _prep_body,
    grid=(_N // _RB,),
    in_specs=[
        pl.BlockSpec((1, _RB, 1), lambda i: (0, i, 0)),
        pl.BlockSpec((1, _RB, 1), lambda i: (1, i, 0)),
        pl.BlockSpec((_RB, _D), lambda i: (i, 0)),
    ],
    out_specs=pl.BlockSpec((_RB, _D), lambda i: (i, 0)),
    out_shape=jax.ShapeDtypeStruct((_N, _D), jnp.float32),
)


def _dense_mid_body(p0_ref, p1_ref, d0_ref, d1_ref, x0_ref, w_ref, b_ref,
                    hs_ref):
    rinv = lax.rsqrt(jnp.maximum(d0_ref[0] + d1_ref[0], 1.0))
    agg = (p0_ref[0] + p1_ref[0]) * rinv
    support = (1.0 - _ALPHA) * agg + _ALPHA * x0_ref[...]
    o = jnp.dot(support, w_ref[...], preferred_element_type=jnp.float32)
    o = jnp.maximum(o + b_ref[...], 0.0)
    hs_ref[...] = o * rinv


def _dense_fin_body(p0_ref, p1_ref, d0_ref, d1_ref, x0_ref, w_ref, b_ref,
                    out_ref):
    rinv = lax.rsqrt(jnp.maximum(d0_ref[0] + d1_ref[0], 1.0))
    agg = (p0_ref[0] + p1_ref[0]) * rinv
    support = (1.0 - _ALPHA) * agg + _ALPHA * x0_ref[...]
    o = jnp.dot(support, w_ref[...], preferred_element_type=jnp.float32)
    out_ref[...] = jnp.maximum(o + b_ref[...], 0.0)


_dense_in_specs = [
    pl.BlockSpec((1, _RB, _D), lambda i: (0, i, 0)),
    pl.BlockSpec((1, _RB, _D), lambda i: (1, i, 0)),
    pl.BlockSpec((1, _RB, 1), lambda i: (0, i, 0)),
    pl.BlockSpec((1, _RB, 1), lambda i: (1, i, 0)),
    pl.BlockSpec((_RB, _D), lambda i: (i, 0)),
    pl.BlockSpec((_D, _D), lambda i: (0, 0)),
    pl.BlockSpec((1, _D), lambda i: (0, 0)),
]

_dense_mid_call = pl.pallas_call(
    _dense_mid_body,
    grid=(_N // _RB,),
    in_specs=_dense_in_specs,
    out_specs=pl.BlockSpec((_RB, _D), lambda i: (i, 0)),
    out_shape=jax.ShapeDtypeStruct((_N, _D), jnp.float32),
)

_dense_fin_call = pl.pallas_call(
    _dense_fin_body,
    grid=(_N // _RB,),
    in_specs=_dense_in_specs,
    out_specs=pl.BlockSpec((_RB, _D), lambda i: (i, 0)),
    out_shape=jax.ShapeDtypeStruct((_N, _D), jnp.float32),
)


def kernel(drug_cell_pair_feature, edge_idx, W, b):
    x0 = drug_cell_pair_feature
    src = edge_idx[0].astype(jnp.int32)
    dst = edge_idx[1].astype(jnp.int32)
    src2d = src.reshape(_NC * _NS * _CPT, _CH)
    dst2d = dst.reshape(_NC * _NS * _CPT, _CH)
    b2 = b.reshape(1, _D)
    zeros = jnp.zeros((_NP, _D), jnp.float32)

    deg = _deg_kernel(dst2d)
    deg3 = deg.reshape(_NC, _NP, 1)
    hs = _prep_call(deg3, deg3, x0)
    for _ in range(4):
        p = _prop_kernel(hs, src2d, dst2d, zeros)
        hs = _dense_mid_call(p, p, deg3, deg3, x0, W, b2)
    p = _prop_kernel(hs, src2d, dst2d, zeros)
    return _dense_fin_call(p, p, deg3, deg3, x0, W, b2)


# speedup vs baseline: 1.2625x; 1.0014x over previous
import functools

import jax
import jax.numpy as jnp
from jax import lax
from jax.experimental import pallas as pl
from jax.experimental.pallas import tpu as pltpu
from jax.experimental.pallas import tpu_sc as plsc

_N = 10000
_E = 320000
_D = 128
_ALPHA = 0.1

_NC = 2
_NS = 16
_NP = 10240
_ROWS_T = _NP // _NS

_CH = 125
_CPT = _E // (_NC * _NS * _CH)

_RB = 5000

_mesh = plsc.VectorSubcoreMesh(core_axis_name="c", subcore_axis_name="s")


@functools.partial(
    pl.kernel,
    out_type=jax.ShapeDtypeStruct((_NC, _NP), jnp.float32),
    mesh=_mesh,
    scratch_types=[
        pltpu.VMEM((_CPT, _CH), jnp.int32),
        pltpu.VMEM((128,), jnp.float32),
        pltpu.VMEM((_ROWS_T,), jnp.float32),
        pltpu.VMEM_SHARED((_NP,), jnp.float32),
        pltpu.SemaphoreType.DMA,
        pltpu.SemaphoreType.DMA,
    ],
)
def _deg_kernel(dst_hbm, deg_hbm, idx_v, ones_v, z_v, deg_sp, dsem_a, dsem_b):
    c = lax.axis_index("c")
    s = lax.axis_index("s")

    def _zero(i, _):
        z_v[pl.ds(i * 16, 16)] = jnp.zeros((16,), jnp.float32)
        return 0

    lax.fori_loop(0, _ROWS_T // 16, _zero, 0)

    def _one(i, _):
        ones_v[pl.ds(i * 16, 16)] = jnp.ones((16,), jnp.float32)
        return 0

    lax.fori_loop(0, 8, _one, 0)

    pltpu.sync_copy(z_v, deg_sp.at[pl.ds(s * _ROWS_T, _ROWS_T)])
    base = (c * _NS + s) * _CPT
    pltpu.sync_copy(dst_hbm.at[pl.ds(base, _CPT)], idx_v)
    plsc.subcore_barrier()

    def _scat(i, _):
        j0 = 2 * i
        pltpu.async_copy(ones_v.at[pl.ds(0, _CH)],
                         deg_sp.at[idx_v.at[j0]], dsem_a, add=True)
        pltpu.async_copy(ones_v.at[pl.ds(0, _CH)],
                         deg_sp.at[idx_v.at[j0 + 1]], dsem_b, add=True)
        pltpu.make_async_copy(ones_v.at[pl.ds(0, _CH)],
                              deg_sp.at[idx_v.at[j0]], dsem_a).wait()
        pltpu.make_async_copy(ones_v.at[pl.ds(0, _CH)],
                              deg_sp.at[idx_v.at[j0]], dsem_b).wait()
        return 0

    lax.fori_loop(0, _CPT // 2, _scat, 0)
    plsc.subcore_barrier()
    pltpu.sync_copy(deg_sp.at[pl.ds(s * _ROWS_T, _ROWS_T)],
                    deg_hbm.at[c, pl.ds(s * _ROWS_T, _ROWS_T)])


@functools.partial(
    pl.kernel,
    out_type=jax.ShapeDtypeStruct((_NC, _NP, _D), jnp.float32),
    mesh=_mesh,
    scratch_types=[
        pltpu.VMEM((_CPT // 2, _CH), jnp.int32),
        pltpu.VMEM((_CPT // 2, _CH), jnp.int32),
        pltpu.VMEM((_CH, _D), jnp.float32),
        pltpu.VMEM((_CH, _D), jnp.float32),
        pltpu.VMEM_SHARED((_NP, _D), jnp.float32),
        pltpu.SemaphoreType.DMA,
        pltpu.SemaphoreType.DMA,
    ],
)
def _prop_kernel(hs_hbm, src_hbm, dst_hbm, zero_hbm, out_hbm,
                 src_v, dst_v, rows_a, rows_b, agg_sp, sem_a, sem_b):
    c = lax.axis_index("c")
    s = lax.axis_index("s")
    half = _CPT // 2

    pltpu.sync_copy(zero_hbm.at[pl.ds(s * _ROWS_T, _ROWS_T)],
                    agg_sp.at[pl.ds(s * _ROWS_T, _ROWS_T)])
    base = (c * _NS + s) * _CPT
    pltpu.sync_copy(src_hbm.at[pl.ds(base, half)], src_v)
    pltpu.sync_copy(dst_hbm.at[pl.ds(base, half)], dst_v)
    plsc.subcore_barrier()

    def _run_half(_):
        pltpu.async_copy(hs_hbm.at[src_v.at[0]], rows_a, sem_a)

        def _body(i, _):
            j0 = 2 * i
            pltpu.async_copy(hs_hbm.at[src_v.at[j0 + 1]], rows_b, sem_b)
            pltpu.make_async_copy(hs_hbm.at[src_v.at[j0]], rows_a, sem_a).wait()
            pltpu.sync_copy(rows_a, agg_sp.at[dst_v.at[j0]], add=True)
            jn = jnp.minimum(j0 + 2, half - 1)
            pltpu.async_copy(hs_hbm.at[src_v.at[jn]], rows_a, sem_a)
            pltpu.make_async_copy(hs_hbm.at[src_v.at[j0 + 1]], rows_b, sem_b).wait()
            pltpu.sync_copy(rows_b, agg_sp.at[dst_v.at[j0 + 1]], add=True)
            return 0

        lax.fori_loop(0, half // 2, _body, 0)
        pltpu.make_async_copy(hs_hbm.at[src_v.at[half - 1]], rows_a, sem_a).wait()

    _run_half(None)
    pltpu.sync_copy(src_hbm.at[pl.ds(base + half, half)], src_v)
    pltpu.sync_copy(dst_hbm.at[pl.ds(base + half, half)], dst_v)
    _run_half(None)
    plsc.subcore_barrier()
    pltpu.sync_copy(agg_sp.at[pl.ds(s * _ROWS_T, _ROWS_T)],
                    out_hbm.at[c, pl.ds(s * _ROWS_T, _ROWS_T)])


def _prep_body(d0_ref, d1_ref, x0_ref, hs_ref):
    rinv = lax.rsqrt(jnp.maximum(d0_ref[0] + d1_ref[0], 1.0))
    hs_ref[...] = x0_ref[...] * rinv


_prep_call = pl.pallas_call(
    _prep_body,
    grid=(_N // _RB,),
    in_specs=[
        pl.BlockSpec((1, _RB, 1), lambda i: (0, i, 0)),
        pl.BlockSpec((1, _RB, 1), lambda i: (1, i, 0)),
        pl.BlockSpec((_RB, _D), lambda i: (i, 0)),
    ],
    out_specs=pl.BlockSpec((_RB, _D), lambda i: (i, 0)),
    out_shape=jax.ShapeDtypeStruct((_N, _D), jnp.float32),
)


def _dense_mid_body(p0_ref, p1_ref, d0_ref, d1_ref, x0_ref, w_ref, b_ref,
                    hs_ref):
    rinv = lax.rsqrt(jnp.maximum(d0_ref[0] + d1_ref[0], 1.0))
    agg = (p0_ref[0] + p1_ref[0]) * rinv
    support = (1.0 - _ALPHA) * agg + _ALPHA * x0_ref[...]
    o = jnp.dot(support, w_ref[...], preferred_element_type=jnp.float32)
    o = jnp.maximum(o + b_ref[...], 0.0)
    hs_ref[...] = o * rinv


def _dense_fin_body(p0_ref, p1_ref, d0_ref, d1_ref, x0_ref, w_ref, b_ref,
                    out_ref):
    rinv = lax.rsqrt(jnp.maximum(d0_ref[0] + d1_ref[0], 1.0))
    agg = (p0_ref[0] + p1_ref[0]) * rinv
    support = (1.0 - _ALPHA) * agg + _ALPHA * x0_ref[...]
    o = jnp.dot(support, w_ref[...], preferred_element_type=jnp.float32)
    out_ref[...] = jnp.maximum(o + b_ref[...], 0.0)


_dense_in_specs = [
    pl.BlockSpec((1, _RB, _D), lambda i: (0, i, 0)),
    pl.BlockSpec((1, _RB, _D), lambda i: (1, i, 0)),
    pl.BlockSpec((1, _RB, 1), lambda i: (0, i, 0)),
    pl.BlockSpec((1, _RB, 1), lambda i: (1, i, 0)),
    pl.BlockSpec((_RB, _D), lambda i: (i, 0)),
    pl.BlockSpec((_D, _D), lambda i: (0, 0)),
    pl.BlockSpec((1, _D), lambda i: (0, 0)),
]

_dense_mid_call = pl.pallas_call(
    _dense_mid_body,
    grid=(_N // _RB,),
    in_specs=_dense_in_specs,
    out_specs=pl.BlockSpec((_RB, _D), lambda i: (i, 0)),
    out_shape=jax.ShapeDtypeStruct((_N, _D), jnp.float32),
)

_dense_fin_call = pl.pallas_call(
    _dense_fin_body,
    grid=(_N // _RB,),
    in_specs=_dense_in_specs,
    out_specs=pl.BlockSpec((_RB, _D), lambda i: (i, 0)),
    out_shape=jax.ShapeDtypeStruct((_N, _D), jnp.float32),
)


def kernel(drug_cell_pair_feature, edge_idx, W, b):
    x0 = drug_cell_pair_feature
    src = edge_idx[0].astype(jnp.int32)
    dst = edge_idx[1].astype(jnp.int32)
    src2d = src.reshape(_NC * _NS * _CPT, _CH)
    dst2d = dst.reshape(_NC * _NS * _CPT, _CH)
    b2 = b.reshape(1, _D)
    zeros = jnp.zeros((_NP, _D), jnp.float32)

    deg = _deg_kernel(dst2d)
    deg3 = deg.reshape(_NC, _NP, 1)
    hs = _prep_call(deg3, deg3, x0)
    for _ in range(4):
        p = _prop_kernel(hs, src2d, dst2d, zeros)
        hs = _dense_mid_call(p, p, deg3, deg3, x0, W, b2)
    p = _prop_kernel(hs, src2d, dst2d, zeros)
    return _dense_fin_call(p, p, deg3, deg3, x0, W, b2)
